# Initial kernel scaffold; baseline (speedup 1.0000x reference)
import jax
import jax.numpy as jnp
from jax.experimental import pallas as pl


def kernel(encoded_sents, mask, keys, U, V, W):
    raise NotImplementedError("write your pallas kernel here")


# speedup vs baseline: 2.3613x; 2.3613x over previous
import jax
import jax.numpy as jnp
from jax.experimental import pallas as pl

B, S, K, D = 1024, 50, 20, 128
BB = 128


def _entity_kernel(x_ref, m_ref, keys_ref, U_ref, V_ref, W_ref, out_ref):
    keys = keys_ref[...]
    U = U_ref[...]
    V = V_ref[...]
    W = W_ref[...]

    keysV = jnp.dot(keys.reshape(BB * K, D), V,
                    preferred_element_type=jnp.float32).reshape(BB, K, D)

    def step(t, h):
        x_t = x_ref[t]
        m_t = m_ref[t].reshape(BB, 1, 1)
        g = jax.nn.sigmoid(
            jnp.sum(x_t[:, None, :] * (h + keys), axis=2))
        hU = jnp.dot(h.reshape(BB * K, D), U,
                     preferred_element_type=jnp.float32).reshape(BB, K, D)
        xW = jnp.dot(x_t, W, preferred_element_type=jnp.float32)
        h_tilda = jax.nn.relu(hU + keysV + xW[:, None, :])
        upd = h + g[..., None] * h_tilda
        denom = jnp.sqrt(jnp.maximum(
            jnp.sum(upd * upd, axis=2, keepdims=True), 1e-12))
        upd = upd / denom
        return h + m_t * (upd - h)

    h0 = jnp.zeros((BB, K, D), dtype=jnp.float32)
    out_ref[...] = jax.lax.fori_loop(0, S, step, h0)


@jax.jit
def kernel(encoded_sents, mask, keys, U, V, W):
    x_t_first = jnp.swapaxes(encoded_sents, 0, 1)
    mask_f = jnp.swapaxes(mask, 0, 1).astype(jnp.float32)[:, None, :]
    grid = (B // BB,)
    return pl.pallas_call(
        _entity_kernel,
        grid=grid,
        in_specs=[
            pl.BlockSpec((S, BB, D), lambda i: (0, i, 0)),
            pl.BlockSpec((S, 1, BB), lambda i: (0, 0, i)),
            pl.BlockSpec((BB, K, D), lambda i: (i, 0, 0)),
            pl.BlockSpec((D, D), lambda i: (0, 0)),
            pl.BlockSpec((D, D), lambda i: (0, 0)),
            pl.BlockSpec((D, D), lambda i: (0, 0)),
        ],
        out_specs=pl.BlockSpec((BB, K, D), lambda i: (i, 0, 0)),
        out_shape=jax.ShapeDtypeStruct((B, K, D), jnp.float32),
    )(x_t_first, mask_f, keys, U, V, W)


# speedup vs baseline: 2.6648x; 1.1285x over previous
import jax
import jax.numpy as jnp
from jax.experimental import pallas as pl

B, S, K, D = 1024, 50, 20, 128
BB = 128


def _entity_kernel(x_ref, m_ref, keys_ref, U_ref, V_ref, W_ref, out_ref):
    keys = keys_ref[...]
    U = U_ref[...]
    V = V_ref[...]
    W = W_ref[...]

    keysV = jnp.dot(keys.reshape(BB * K, D), V,
                    preferred_element_type=jnp.float32).reshape(BB, K, D)

    def step(t, h):
        x_t = x_ref[t]
        m_t = m_ref[t].reshape(BB, 1)
        g = m_t * jax.nn.sigmoid(
            jnp.sum(x_t[:, None, :] * (h + keys), axis=2))
        hU = jnp.dot(h.reshape(BB * K, D), U,
                     preferred_element_type=jnp.float32).reshape(BB, K, D)
        xW = jnp.dot(x_t, W, preferred_element_type=jnp.float32)
        h_tilda = jax.nn.relu(hU + keysV + xW[:, None, :])
        upd = h + g[..., None] * h_tilda
        inv = jax.lax.rsqrt(jnp.maximum(
            jnp.sum(upd * upd, axis=2, keepdims=True), 1e-12))
        return upd * inv

    h0 = jnp.zeros((BB, K, D), dtype=jnp.float32)
    out_ref[...] = jax.lax.fori_loop(0, S, step, h0)


@jax.jit
def kernel(encoded_sents, mask, keys, U, V, W):
    x_t_first = jnp.swapaxes(encoded_sents, 0, 1)
    mask_f = jnp.swapaxes(mask, 0, 1).astype(jnp.float32)[:, None, :]
    grid = (B // BB,)
    return pl.pallas_call(
        _entity_kernel,
        grid=grid,
        in_specs=[
            pl.BlockSpec((S, BB, D), lambda i: (0, i, 0)),
            pl.BlockSpec((S, 1, BB), lambda i: (0, 0, i)),
            pl.BlockSpec((BB, K, D), lambda i: (i, 0, 0)),
            pl.BlockSpec((D, D), lambda i: (0, 0)),
            pl.BlockSpec((D, D), lambda i: (0, 0)),
            pl.BlockSpec((D, D), lambda i: (0, 0)),
        ],
        out_specs=pl.BlockSpec((BB, K, D), lambda i: (i, 0, 0)),
        out_shape=jax.ShapeDtypeStruct((B, K, D), jnp.float32),
    )(x_t_first, mask_f, keys, U, V, W)
